# Initial kernel scaffold; baseline (speedup 1.0000x reference)
import jax
import jax.numpy as jnp
from jax.experimental import pallas as pl


def kernel(table, W1, b1, w_path, Wi1, bi1, Wi2, bi2, locs_gene, locs_combos):
    raise NotImplementedError("write your pallas kernel here")


# speedup vs baseline: 1.5349x; 1.5349x over previous
import functools

import jax
import jax.numpy as jnp
import numpy as np
from jax import lax
from jax.experimental import pallas as pl
from jax.experimental.pallas import tpu as pltpu
from jax.experimental.pallas import tpu_sc as plsc

G, D, P = 20000, 128, 16
H = 128
PCA = 512
B = 16384
N = 32768

HP = H * P
CSLICE = 128
NSLICES = HP // CSLICE
NC, NS, L = 2, 16, 16
B2 = B // NC
N_PER_TILE = N // NS
CHUNK = 128
NCHUNK = N_PER_TILE // CHUNK
ROWS_PER_TILE = B2 // NS


def _table_transform_body(x_ref, w_ref, b_ref, th_ref, m_ref):
    x = x_ref[...]
    xt = jnp.transpose(x, (0, 2, 1))
    tg = x.shape[0]
    x2 = xt.reshape(tg * P, D)
    y = jnp.dot(x2, w_ref[...], preferred_element_type=jnp.float32)
    y = y + b_ref[...]
    y = jnp.where(y >= 0, y, 0.01 * y)
    th_ref[...] = jnp.transpose(y.reshape(tg, P, H), (0, 2, 1))
    s = jnp.sum(xt, axis=2)
    m = (s != 0).astype(jnp.float32)
    m_ref[...] = jnp.concatenate(
        [m, jnp.zeros((tg, H - P), jnp.float32)], axis=1)


def _table_transform(table, W1, b1r):
    TG = 200
    grid = G // TG
    return pl.pallas_call(
        _table_transform_body,
        grid=(grid,),
        in_specs=[
            pl.BlockSpec((TG, D, P), lambda i: (i, 0, 0)),
            pl.BlockSpec((D, H), lambda i: (0, 0)),
            pl.BlockSpec((1, H), lambda i: (0, 0)),
        ],
        out_specs=[
            pl.BlockSpec((TG, H, P), lambda i: (i, 0, 0)),
            pl.BlockSpec((TG, H), lambda i: (i, 0)),
        ],
        out_shape=[
            jax.ShapeDtypeStruct((G, H, P), jnp.float32),
            jax.ShapeDtypeStruct((G, H), jnp.float32),
        ],
    )(table, W1, b1r)


def _seg_body(th_hbm, mg_hbm, lg_hbm, lc_hbm, agg_hbm, magg_hbm,
              lg_v, lc_v, lcx_v, gidx, gbuf, zbuf, acc, sem):
    c = lax.axis_index("c")
    s = lax.axis_index("s")
    row0 = s * NCHUNK
    arow0 = s * ROWS_PER_TILE
    lo = c * B2

    pltpu.sync_copy(lg_hbm.at[pl.ds(row0, NCHUNK)], lg_v)
    pltpu.sync_copy(lc_hbm.at[pl.ds(row0, NCHUNK)], lc_v)

    def _lx(k, carry):
        r = k // (CHUNK // L)
        q = (k % (CHUNK // L)) * L
        v = lc_v[r, pl.ds(q, L)] - lo
        inr = (v >= 0) & (v < B2)
        lcx_v[r, pl.ds(q, L)] = jnp.where(inr, v, B2)
        return carry
    lax.fori_loop(0, NCHUNK * (CHUNK // L), _lx, 0)

    def _zb(k, carry):
        zbuf[k // (CSLICE // L), pl.ds((k % (CSLICE // L)) * L, L)] = (
            jnp.zeros((L,), jnp.float32))
        return carry
    lax.fori_loop(0, CHUNK * (CSLICE // L), _zb, 0)

    def _z(i, carry):
        pltpu.sync_copy(zbuf, acc.at[pl.ds(arow0 + i * CHUNK, CHUNK)])
        return carry
    lax.fori_loop(0, ROWS_PER_TILE // CHUNK, _z, 0)

    @pl.when(s == 0)
    def _zdump():
        pltpu.sync_copy(zbuf.at[pl.ds(0, 8)], acc.at[pl.ds(B2, 8)])

    plsc.subcore_barrier()

    def _slice(sg, carry):
        def _gx(k, carry2):
            r = k // (CHUNK // L)
            q = (k % (CHUNK // L)) * L
            gidx[r, pl.ds(q, L)] = lg_v[r, pl.ds(q, L)] * NSLICES + sg
            return carry2
        lax.fori_loop(0, NCHUNK * (CHUNK // L), _gx, 0)

        def _chunk(cq, carry2):
            pltpu.async_copy(th_hbm.at[gidx.at[cq]], gbuf, sem).wait()
            pltpu.sync_copy(gbuf, acc.at[lcx_v.at[cq]], add=True)
            return carry2
        lax.fori_loop(0, NCHUNK, _chunk, 0)

        plsc.subcore_barrier()

        def _wo(i, carry2):
            r = arow0 + i * CHUNK
            pltpu.sync_copy(acc.at[pl.ds(r, CHUNK)],
                            agg_hbm.at[pl.ds(lo + r, CHUNK),
                                       pl.ds(sg * CSLICE, CSLICE)])
            pltpu.sync_copy(zbuf, acc.at[pl.ds(r, CHUNK)])
            return carry2
        lax.fori_loop(0, ROWS_PER_TILE // CHUNK, _wo, 0)

        plsc.subcore_barrier()
        return carry
    lax.fori_loop(0, NSLICES, _slice, 0)

    def _mchunk(cq, carry):
        pltpu.async_copy(mg_hbm.at[lg_v.at[cq]], gbuf, sem).wait()
        pltpu.sync_copy(gbuf, acc.at[lcx_v.at[cq]], add=True)
        return carry
    lax.fori_loop(0, NCHUNK, _mchunk, 0)
    plsc.subcore_barrier()

    def _mwo(i, carry):
        r = arow0 + i * CHUNK
        pltpu.sync_copy(acc.at[pl.ds(r, CHUNK)],
                        magg_hbm.at[pl.ds(lo + r, CHUNK)])
        return carry
    lax.fori_loop(0, ROWS_PER_TILE // CHUNK, _mwo, 0)


def _segment_aggregate(th_rows, maskg, lg2, lc2):
    f = pl.kernel(
        _seg_body,
        out_type=[
            jax.ShapeDtypeStruct((B, HP), jnp.float32),
            jax.ShapeDtypeStruct((B, H), jnp.float32),
        ],
        mesh=plsc.VectorSubcoreMesh(core_axis_name="c", subcore_axis_name="s"),
        scratch_types=[
            pltpu.VMEM((NCHUNK, CHUNK), jnp.int32),
            pltpu.VMEM((NCHUNK, CHUNK), jnp.int32),
            pltpu.VMEM((NCHUNK, CHUNK), jnp.int32),
            pltpu.VMEM((NCHUNK, CHUNK), jnp.int32),
            pltpu.VMEM((CHUNK, CSLICE), jnp.float32),
            pltpu.VMEM((CHUNK, CSLICE), jnp.float32),
            pltpu.VMEM_SHARED((B2 + 8, CSLICE), jnp.float32),
            pltpu.SemaphoreType.DMA,
        ],
    )
    return f(th_rows, maskg, lg2, lc2)


def _pool_mlp_body(ag_ref, mg_ref, wp_ref, t_ref, s_ref,
                   wi1_ref, bi1_ref, wi2_ref, bi2_ref, out_ref):
    wp = wp_ref[...]
    m = jnp.max(wp, axis=1, keepdims=True)
    e = jnp.exp(wp - m)
    a = e / jnp.sum(e, axis=1, keepdims=True)
    mg = mg_ref[...][:, :P]
    wm = (mg > 0).astype(jnp.float32) * a
    wmbig = jnp.dot(wm, t_ref[...], preferred_element_type=jnp.float32)
    y = ag_ref[...] * wmbig
    pooled = jnp.dot(y, s_ref[...], preferred_element_type=jnp.float32)
    h1 = jnp.dot(pooled, wi1_ref[...], preferred_element_type=jnp.float32)
    h1 = h1 + bi1_ref[...]
    h1 = jnp.where(h1 >= 0, h1, 0.01 * h1)
    o = jnp.dot(h1, wi2_ref[...], preferred_element_type=jnp.float32)
    out_ref[...] = o + bi2_ref[...]


_TILE_T = np.tile(np.eye(P, dtype=np.float32), (1, H))
_SEL_S = np.kron(np.eye(H, dtype=np.float32), np.ones((P, 1), np.float32))


def _pool_mlp(agg, magg, wpr, Wi1, bi1r, Wi2, bi2r):
    TB = 256
    grid = B // TB
    return pl.pallas_call(
        _pool_mlp_body,
        grid=(grid,),
        in_specs=[
            pl.BlockSpec((TB, HP), lambda i: (i, 0)),
            pl.BlockSpec((TB, H), lambda i: (i, 0)),
            pl.BlockSpec((1, P), lambda i: (0, 0)),
            pl.BlockSpec((P, HP), lambda i: (0, 0)),
            pl.BlockSpec((HP, H), lambda i: (0, 0)),
            pl.BlockSpec((H, H), lambda i: (0, 0)),
            pl.BlockSpec((1, H), lambda i: (0, 0)),
            pl.BlockSpec((H, PCA), lambda i: (0, 0)),
            pl.BlockSpec((1, PCA), lambda i: (0, 0)),
        ],
        out_specs=pl.BlockSpec((TB, PCA), lambda i: (i, 0)),
        out_shape=jax.ShapeDtypeStruct((B, PCA), jnp.float32),
    )(agg, magg, wpr, jnp.asarray(_TILE_T), jnp.asarray(_SEL_S),
      Wi1, bi1r, Wi2, bi2r)


def kernel(table, W1, b1, w_path, Wi1, bi1, Wi2, bi2, locs_gene, locs_combos):
    th, maskg = _table_transform(table, W1, b1.reshape(1, H))
    th_rows = th.reshape(G * NSLICES, CSLICE)
    lg2 = locs_gene.reshape(N // CHUNK, CHUNK)
    lc2 = locs_combos.reshape(N // CHUNK, CHUNK)
    agg, magg = _segment_aggregate(th_rows, maskg, lg2, lc2)
    out = _pool_mlp(agg, magg, w_path.reshape(1, P), Wi1,
                    bi1.reshape(1, H), Wi2, bi2.reshape(1, PCA))
    return (out, agg.reshape(B, H, P))


# speedup vs baseline: 4.9707x; 3.2385x over previous
import functools

import jax
import jax.numpy as jnp
import numpy as np
from jax import lax
from jax.experimental import pallas as pl
from jax.experimental.pallas import tpu as pltpu
from jax.experimental.pallas import tpu_sc as plsc

G, D, P = 20000, 128, 16
H = 128
PCA = 512
B = 16384
N = 32768

HP = H * P
CSLICE = 128
NSLICES = HP // CSLICE
NC, NS, L = 2, 16, 16
B2 = B // NC
N_PER_TILE = N // NS
CHUNK = 128
NCHUNK = N_PER_TILE // CHUNK
ROWS_PER_TILE = B2 // NS


def _table_transform_body(x_ref, w_ref, b_ref, th_ref, m_ref):
    x = x_ref[...]
    tg = x.shape[0]
    x2 = x.reshape(tg * P, D)
    y = jnp.dot(x2, w_ref[...], preferred_element_type=jnp.float32)
    y = y + b_ref[...]
    y = jnp.where(y >= 0, y, 0.01 * y)
    th_ref[...] = y.reshape(tg, P, H)
    s = jnp.sum(x, axis=2)
    m = (s != 0).astype(jnp.float32)
    m_ref[...] = jnp.concatenate(
        [m, jnp.zeros((tg, H - P), jnp.float32)], axis=1)


def _table_transform(tt, W1, b1r):
    TG = 400
    grid = G // TG
    return pl.pallas_call(
        _table_transform_body,
        grid=(grid,),
        in_specs=[
            pl.BlockSpec((TG, P, D), lambda i: (i, 0, 0)),
            pl.BlockSpec((D, H), lambda i: (0, 0)),
            pl.BlockSpec((1, H), lambda i: (0, 0)),
        ],
        out_specs=[
            pl.BlockSpec((TG, P, H), lambda i: (i, 0, 0)),
            pl.BlockSpec((TG, H), lambda i: (i, 0)),
        ],
        out_shape=[
            jax.ShapeDtypeStruct((G, P, H), jnp.float32),
            jax.ShapeDtypeStruct((G, H), jnp.float32),
        ],
    )(tt, W1, b1r)


def _seg_body(th_hbm, mg_hbm, lg_hbm, lc_hbm, agg_hbm, magg_hbm,
              lg_v, lc_v, lcx_v, gidx, gbuf, zbuf, acc, sem):
    c = lax.axis_index("c")
    s = lax.axis_index("s")
    row0 = s * NCHUNK
    arow0 = s * ROWS_PER_TILE
    lo = c * B2

    pltpu.sync_copy(lg_hbm.at[pl.ds(row0, NCHUNK)], lg_v)
    pltpu.sync_copy(lc_hbm.at[pl.ds(row0, NCHUNK)], lc_v)

    def _lx(k, carry):
        r = k // (CHUNK // L)
        q = (k % (CHUNK // L)) * L
        v = lc_v[r, pl.ds(q, L)] - lo
        inr = (v >= 0) & (v < B2)
        lcx_v[r, pl.ds(q, L)] = jnp.where(inr, v, B2)
        return carry
    lax.fori_loop(0, NCHUNK * (CHUNK // L), _lx, 0)

    def _zb(k, carry):
        zbuf[k // (CSLICE // L), pl.ds((k % (CSLICE // L)) * L, L)] = (
            jnp.zeros((L,), jnp.float32))
        return carry
    lax.fori_loop(0, CHUNK * (CSLICE // L), _zb, 0)

    def _z(i, carry):
        pltpu.sync_copy(zbuf, acc.at[pl.ds(arow0 + i * CHUNK, CHUNK)])
        return carry
    lax.fori_loop(0, ROWS_PER_TILE // CHUNK, _z, 0)

    @pl.when(s == 0)
    def _zdump():
        pltpu.sync_copy(zbuf.at[pl.ds(0, 8)], acc.at[pl.ds(B2, 8)])

    plsc.subcore_barrier()

    def _slice(sg, carry):
        def _gx(k, carry2):
            r = k // (CHUNK // L)
            q = (k % (CHUNK // L)) * L
            gidx[r, pl.ds(q, L)] = lg_v[r, pl.ds(q, L)] * NSLICES + sg
            return carry2
        lax.fori_loop(0, NCHUNK * (CHUNK // L), _gx, 0)

        def _chunk(cq, carry2):
            pltpu.async_copy(th_hbm.at[gidx.at[cq]], gbuf, sem).wait()
            pltpu.sync_copy(gbuf, acc.at[lcx_v.at[cq]], add=True)
            return carry2
        lax.fori_loop(0, NCHUNK, _chunk, 0)

        plsc.subcore_barrier()

        def _wo(i, carry2):
            r = arow0 + i * CHUNK
            pltpu.sync_copy(acc.at[pl.ds(r, CHUNK)],
                            agg_hbm.at[pl.ds(lo + r, CHUNK),
                                       pl.ds(sg * CSLICE, CSLICE)])
            pltpu.sync_copy(zbuf, acc.at[pl.ds(r, CHUNK)])
            return carry2
        lax.fori_loop(0, ROWS_PER_TILE // CHUNK, _wo, 0)

        plsc.subcore_barrier()
        return carry
    lax.fori_loop(0, NSLICES, _slice, 0)

    def _mchunk(cq, carry):
        pltpu.async_copy(mg_hbm.at[lg_v.at[cq]], gbuf, sem).wait()
        pltpu.sync_copy(gbuf, acc.at[lcx_v.at[cq]], add=True)
        return carry
    lax.fori_loop(0, NCHUNK, _mchunk, 0)
    plsc.subcore_barrier()

    def _mwo(i, carry):
        r = arow0 + i * CHUNK
        pltpu.sync_copy(acc.at[pl.ds(r, CHUNK)],
                        magg_hbm.at[pl.ds(lo + r, CHUNK)])
        return carry
    lax.fori_loop(0, ROWS_PER_TILE // CHUNK, _mwo, 0)


def _segment_aggregate(th_rows, maskg, lg2, lc2):
    f = pl.kernel(
        _seg_body,
        out_type=[
            jax.ShapeDtypeStruct((B, HP), jnp.float32),
            jax.ShapeDtypeStruct((B, H), jnp.float32),
        ],
        mesh=plsc.VectorSubcoreMesh(core_axis_name="c", subcore_axis_name="s"),
        scratch_types=[
            pltpu.VMEM((NCHUNK, CHUNK), jnp.int32),
            pltpu.VMEM((NCHUNK, CHUNK), jnp.int32),
            pltpu.VMEM((NCHUNK, CHUNK), jnp.int32),
            pltpu.VMEM((NCHUNK, CHUNK), jnp.int32),
            pltpu.VMEM((CHUNK, CSLICE), jnp.float32),
            pltpu.VMEM((CHUNK, CSLICE), jnp.float32),
            pltpu.VMEM_SHARED((B2 + 8, CSLICE), jnp.float32),
            pltpu.SemaphoreType.DMA,
        ],
    )
    return f(th_rows, maskg, lg2, lc2)


def _pool_mlp_body(ag_ref, mg_ref, wp_ref, t_ref, s_ref,
                   wi1_ref, bi1_ref, wi2_ref, bi2_ref, out_ref):
    wp = wp_ref[...]
    m = jnp.max(wp, axis=1, keepdims=True)
    e = jnp.exp(wp - m)
    a = e / jnp.sum(e, axis=1, keepdims=True)
    mg = mg_ref[...][:, :P]
    wm = (mg > 0).astype(jnp.float32) * a
    wmbig = jnp.dot(wm, t_ref[...], preferred_element_type=jnp.float32)
    y = ag_ref[...] * wmbig
    pooled = jnp.dot(y, s_ref[...], preferred_element_type=jnp.float32)
    h1 = jnp.dot(pooled, wi1_ref[...], preferred_element_type=jnp.float32)
    h1 = h1 + bi1_ref[...]
    h1 = jnp.where(h1 >= 0, h1, 0.01 * h1)
    o = jnp.dot(h1, wi2_ref[...], preferred_element_type=jnp.float32)
    out_ref[...] = o + bi2_ref[...]


_TILE_T = np.kron(np.eye(P, dtype=np.float32), np.ones((1, H), np.float32))
_SEL_S = np.tile(np.eye(H, dtype=np.float32), (P, 1))


def _pool_mlp(agg, magg, wpr, Wi1, bi1r, Wi2, bi2r):
    TB = 256
    grid = B // TB
    return pl.pallas_call(
        _pool_mlp_body,
        grid=(grid,),
        in_specs=[
            pl.BlockSpec((TB, HP), lambda i: (i, 0)),
            pl.BlockSpec((TB, H), lambda i: (i, 0)),
            pl.BlockSpec((1, P), lambda i: (0, 0)),
            pl.BlockSpec((P, HP), lambda i: (0, 0)),
            pl.BlockSpec((HP, H), lambda i: (0, 0)),
            pl.BlockSpec((H, H), lambda i: (0, 0)),
            pl.BlockSpec((1, H), lambda i: (0, 0)),
            pl.BlockSpec((H, PCA), lambda i: (0, 0)),
            pl.BlockSpec((1, PCA), lambda i: (0, 0)),
        ],
        out_specs=pl.BlockSpec((TB, PCA), lambda i: (i, 0)),
        out_shape=jax.ShapeDtypeStruct((B, PCA), jnp.float32),
    )(agg, magg, wpr, jnp.asarray(_TILE_T), jnp.asarray(_SEL_S),
      Wi1, bi1r, Wi2, bi2r)


def kernel(table, W1, b1, w_path, Wi1, bi1, Wi2, bi2, locs_gene, locs_combos):
    tt = jnp.swapaxes(table, 1, 2)
    th, maskg = _table_transform(tt, W1, b1.reshape(1, H))
    th_rows = th.reshape(G * NSLICES, CSLICE)
    lg2 = locs_gene.reshape(N // CHUNK, CHUNK)
    lc2 = locs_combos.reshape(N // CHUNK, CHUNK)
    agg, magg = _segment_aggregate(th_rows, maskg, lg2, lc2)
    out = _pool_mlp(agg, magg, w_path.reshape(1, P), Wi1,
                    bi1.reshape(1, H), Wi2, bi2.reshape(1, PCA))
    return (out, jnp.swapaxes(agg.reshape(B, P, H), 1, 2))


# speedup vs baseline: 6.6671x; 1.3413x over previous
import functools

import jax
import jax.numpy as jnp
import numpy as np
from jax import lax
from jax.experimental import pallas as pl
from jax.experimental.pallas import tpu as pltpu
from jax.experimental.pallas import tpu_sc as plsc

G, D, P = 20000, 128, 16
H = 128
PCA = 512
B = 16384
N = 32768

HP = H * P
NC, NS, L = 2, 16, 16
B2 = B // NC
SEG = 512
NPASS = B // SEG
WCAP = 2048
GROUP = 16
ROWS_TILE = SEG // NS
NBOUND = 48


def _table_transform_body(x_ref, w_ref, b_ref, th_ref, m_ref):
    x = x_ref[...]
    tg = x.shape[0]
    x2 = x.reshape(tg * P, D)
    y = jnp.dot(x2, w_ref[...], preferred_element_type=jnp.float32)
    y = y + b_ref[...]
    y = jnp.where(y >= 0, y, 0.01 * y)
    th_ref[...] = y.reshape(tg, P, H)
    s = jnp.sum(x, axis=2)
    m = (s != 0).astype(jnp.float32)
    m_ref[...] = jnp.concatenate(
        [m, jnp.zeros((tg, H - P), jnp.float32)], axis=1)


def _table_transform(tt, W1, b1r):
    TG = 400
    grid = G // TG
    return pl.pallas_call(
        _table_transform_body,
        grid=(grid,),
        in_specs=[
            pl.BlockSpec((TG, P, D), lambda i: (i, 0, 0)),
            pl.BlockSpec((D, H), lambda i: (0, 0)),
            pl.BlockSpec((1, H), lambda i: (0, 0)),
        ],
        out_specs=[
            pl.BlockSpec((TG, P, H), lambda i: (i, 0, 0)),
            pl.BlockSpec((TG, H), lambda i: (i, 0)),
        ],
        out_shape=[
            jax.ShapeDtypeStruct((G, P, H), jnp.float32),
            jax.ShapeDtypeStruct((G, H), jnp.float32),
        ],
    )(tt, W1, b1r)


def _seg_body(th_hbm, mg_hbm, lg_hbm, lc_hbm, bnd_hbm, agg_hbm, magg_hbm,
              lgb, lcb, gidx, lcx, gbuf, mbuf, zbuf, zmb, bnd_v, acc, macc,
              sem):
    c = lax.axis_index("c")
    s = lax.axis_index("s")

    pltpu.sync_copy(bnd_hbm, bnd_v)

    def _zb(k, carry):
        zbuf[k // (P * 8), (k // 8) % P, pl.ds((k % 8) * L, L)] = (
            jnp.zeros((L,), jnp.float32))
        return carry
    lax.fori_loop(0, 4 * P * 8, _zb, 0)

    def _zm(k, carry):
        zmb[k // 8, pl.ds((k % 8) * L, L)] = jnp.zeros((L,), jnp.float32)
        return carry
    lax.fori_loop(0, ROWS_TILE * 8, _zm, 0)

    def _zero_mine():
        for t in range(ROWS_TILE // 4):
            pltpu.sync_copy(zbuf, acc.at[pl.ds(s * ROWS_TILE + t * 4, 4)])
        pltpu.sync_copy(zmb, macc.at[pl.ds(s * ROWS_TILE, ROWS_TILE)])

    _zero_mine()

    @pl.when(s == 0)
    def _zdump():
        pltpu.sync_copy(zbuf, acc.at[pl.ds(SEG, 4)])
        pltpu.sync_copy(zbuf, acc.at[pl.ds(SEG + 4, 4)])
        pltpu.sync_copy(zmb.at[pl.ds(0, 8)], macc.at[pl.ds(SEG, 8)])

    plsc.subcore_barrier()

    def _pass(j, carry):
        i = c * (NPASS // NC) + j
        seg0 = i * SEG
        vb = bnd_v[pl.ds(i, L)]
        n0 = vb[0]
        n1 = vb[1]
        base0 = (n0 // 8) * 8
        span = n1 - base0
        nwin = (span + WCAP - 1) // WCAP

        def _win(w, carry2):
            off = base0 + w * WCAP
            pltpu.sync_copy(lg_hbm.at[pl.ds(off, WCAP)], lgb)
            pltpu.sync_copy(lc_hbm.at[pl.ds(off, WCAP)], lcb)
            lim = jnp.minimum(WCAP, n1 - off)
            ngroups = (lim + GROUP - 1) // GROUP

            def _grp(k, carry3):
                g = s + k * NS
                for t in range(GROUP // L):
                    vlg = lgb[pl.ds(g * GROUP + t * L, L)]
                    vlc = lcb[pl.ds(g * GROUP + t * L, L)]
                    loc = vlc - seg0
                    ok = (loc >= 0) & (loc < SEG)
                    gidx[0, pl.ds(t * L, L)] = vlg
                    lcx[0, pl.ds(t * L, L)] = jnp.where(ok, loc, SEG)
                pltpu.async_copy(th_hbm.at[gidx.at[0]], gbuf, sem).wait()
                pltpu.sync_copy(gbuf, acc.at[lcx.at[0]], add=True)
                pltpu.async_copy(mg_hbm.at[gidx.at[0]], mbuf, sem).wait()
                pltpu.sync_copy(mbuf, macc.at[lcx.at[0]], add=True)
                return carry3
            lax.fori_loop(0, (ngroups - s + NS - 1) // NS, _grp, 0)
            return carry2
        lax.fori_loop(0, nwin, _win, 0)

        plsc.subcore_barrier()

        pltpu.sync_copy(acc.at[pl.ds(s * ROWS_TILE, ROWS_TILE)],
                        agg_hbm.at[pl.ds(seg0 + s * ROWS_TILE, ROWS_TILE)])
        pltpu.sync_copy(macc.at[pl.ds(s * ROWS_TILE, ROWS_TILE)],
                        magg_hbm.at[pl.ds(seg0 + s * ROWS_TILE, ROWS_TILE)])
        _zero_mine()

        @pl.when(s == 0)
        def _zdump2():
            pltpu.sync_copy(zbuf, acc.at[pl.ds(SEG, 4)])
            pltpu.sync_copy(zbuf, acc.at[pl.ds(SEG + 4, 4)])
            pltpu.sync_copy(zmb.at[pl.ds(0, 8)], macc.at[pl.ds(SEG, 8)])

        plsc.subcore_barrier()
        return carry
    lax.fori_loop(0, NPASS // NC, _pass, 0)


def _segment_aggregate(th3, maskg, lg1, lc1, bounds):
    f = pl.kernel(
        _seg_body,
        out_type=[
            jax.ShapeDtypeStruct((B, P, H), jnp.float32),
            jax.ShapeDtypeStruct((B, H), jnp.float32),
        ],
        mesh=plsc.VectorSubcoreMesh(core_axis_name="c", subcore_axis_name="s"),
        scratch_types=[
            pltpu.VMEM((WCAP,), jnp.int32),
            pltpu.VMEM((WCAP,), jnp.int32),
            pltpu.VMEM((1, GROUP), jnp.int32),
            pltpu.VMEM((1, GROUP), jnp.int32),
            pltpu.VMEM((GROUP, P, H), jnp.float32),
            pltpu.VMEM((GROUP, H), jnp.float32),
            pltpu.VMEM((4, P, H), jnp.float32),
            pltpu.VMEM((ROWS_TILE, H), jnp.float32),
            pltpu.VMEM((NBOUND,), jnp.int32),
            pltpu.VMEM_SHARED((SEG + 8, P, H), jnp.float32),
            pltpu.VMEM_SHARED((SEG + 8, H), jnp.float32),
            pltpu.SemaphoreType.DMA,
        ],
    )
    return f(th3, maskg, lg1, lc1, bounds)


def _pool_mlp_body(ag_ref, mg_ref, wp_ref, wi1_ref, bi1_ref, wi2_ref,
                   bi2_ref, out_ref):
    wp = wp_ref[...]
    m = jnp.max(wp, axis=1, keepdims=True)
    e = jnp.exp(wp - m)
    a = e / jnp.sum(e, axis=1, keepdims=True)
    mg = mg_ref[...][:, :P]
    wm = (mg > 0).astype(jnp.float32) * a
    y = ag_ref[...] * wm[:, :, None]
    pooled = jnp.sum(y, axis=1)
    h1 = jnp.dot(pooled, wi1_ref[...], preferred_element_type=jnp.float32)
    h1 = h1 + bi1_ref[...]
    h1 = jnp.where(h1 >= 0, h1, 0.01 * h1)
    o = jnp.dot(h1, wi2_ref[...], preferred_element_type=jnp.float32)
    out_ref[...] = o + bi2_ref[...]


def _pool_mlp(agg3, magg, wpr, Wi1, bi1r, Wi2, bi2r):
    TB = 256
    grid = B // TB
    return pl.pallas_call(
        _pool_mlp_body,
        grid=(grid,),
        in_specs=[
            pl.BlockSpec((TB, P, H), lambda i: (i, 0, 0)),
            pl.BlockSpec((TB, H), lambda i: (i, 0)),
            pl.BlockSpec((1, P), lambda i: (0, 0)),
            pl.BlockSpec((H, H), lambda i: (0, 0)),
            pl.BlockSpec((1, H), lambda i: (0, 0)),
            pl.BlockSpec((H, PCA), lambda i: (0, 0)),
            pl.BlockSpec((1, PCA), lambda i: (0, 0)),
        ],
        out_specs=pl.BlockSpec((TB, PCA), lambda i: (i, 0)),
        out_shape=jax.ShapeDtypeStruct((B, PCA), jnp.float32),
    )(agg3, magg, wpr, Wi1, bi1r, Wi2, bi2r)


def kernel(table, W1, b1, w_path, Wi1, bi1, Wi2, bi2, locs_gene, locs_combos):
    tt = jnp.swapaxes(table, 1, 2)
    th3, maskg = _table_transform(tt, W1, b1.reshape(1, H))
    lg1 = jnp.concatenate([locs_gene, jnp.zeros((WCAP,), jnp.int32)])
    lc1 = jnp.concatenate([locs_combos, jnp.full((WCAP,), B, jnp.int32)])
    bounds = jnp.searchsorted(locs_combos,
                              jnp.arange(0, B + 1, SEG)).astype(jnp.int32)
    bounds = jnp.concatenate(
        [bounds, jnp.full((NBOUND - NPASS - 1,), N, jnp.int32)])
    agg3, magg = _segment_aggregate(th3, maskg, lg1, lc1, bounds)
    out = _pool_mlp(agg3, magg, w_path.reshape(1, P), Wi1,
                    bi1.reshape(1, H), Wi2, bi2.reshape(1, PCA))
    return (out, jnp.swapaxes(agg3, 1, 2))
